# baseline (speedup 1.0000x reference)
import functools

import jax
import jax.numpy as jnp
from jax import lax
from jax.experimental import pallas as pl
from jax.experimental.pallas import tpu as pltpu
from jax.experimental.pallas import tpu_sc as plsc


TM = 128
TF = 1024


def _fused_moe_body(ft_ref, nt_ref, st_ref, en_ref,
                    x_ref, w_ref, wg_ref, wu_ref, wd_ref, out_ref,
                    wg_bf, wu_bf, wd_bf):
    e = pl.program_id(0)
    f = pl.program_id(1)

    @pl.when((e == 0) & (f == 0))
    def _():
        out_ref[...] = jnp.zeros_like(out_ref)

    wg_bf[...] = wg_ref[0].astype(jnp.bfloat16)
    wu_bf[...] = wu_ref[0].astype(jnp.bfloat16)
    wd_bf[...] = wd_ref[0].astype(jnp.bfloat16)

    start = st_ref[e]
    end = en_ref[e]
    first = ft_ref[e]

    def chunk(c, carry):
        base = (first + c) * TM
        x = x_ref[pl.ds(base, TM), :]
        gate = jnp.dot(x, wg_bf[...], preferred_element_type=jnp.float32)
        up = jnp.dot(x, wu_bf[...], preferred_element_type=jnp.float32)
        fused = gate * jax.lax.logistic(gate) * up
        fused = fused * w_ref[pl.ds(base, TM), :]
        part = jnp.dot(fused.astype(jnp.bfloat16), wd_bf[...],
                       preferred_element_type=jnp.float32)
        row = base + jax.lax.broadcasted_iota(jnp.int32, (TM, 1), 0)
        mask = (row >= start) & (row < end)
        out_ref[pl.ds(base, TM), :] += jnp.where(mask, part, 0.0)
        return carry

    jax.lax.fori_loop(0, nt_ref[e], chunk, 0)


@functools.cache
def _make_prep(T, D, K, E, M, SL):
    info = plsc.get_sparse_core_info()
    NW = info.num_cores * info.num_subcores
    L = info.num_lanes
    CT = T // NW
    CS = CT * K
    NV = M // L
    TM_ = TM

    mesh = plsc.VectorSubcoreMesh(core_axis_name="c", subcore_axis_name="s")

    @functools.partial(
        pl.kernel, mesh=mesh,
        out_type=(
            jax.ShapeDtypeStruct((M, SL * 128), jnp.int32),
            jax.ShapeDtypeStruct((M,), jnp.float32),
            jax.ShapeDtypeStruct((T,), jnp.int32),
            jax.ShapeDtypeStruct((T,), jnp.int32),
            jax.ShapeDtypeStruct((4, L), jnp.int32),
        ),
        scratch_types=[
            pltpu.VMEM((M,), jnp.int32),
            pltpu.VMEM((CS,), jnp.float32),
            pltpu.VMEM((CT, SL * 128), jnp.int32),
            pltpu.VMEM((CS,), jnp.int32),
            pltpu.VMEM((CT,), jnp.int32),
            pltpu.VMEM((CT,), jnp.int32),
            pltpu.VMEM((4, L), jnp.int32),
            pltpu.SMEM((E,), jnp.int32),
            pltpu.SemaphoreType.DMA,
            pltpu.SemaphoreType.DMA,
            pltpu.SemaphoreType.DMA,
        ],
        compiler_params=pltpu.CompilerParams(needs_layout_passes=False))
    def prep(x3_hbm, sel_hbm, rw_hbm, xs_hbm, ws_hbm, pe_hbm, po_hbm, meta_hbm,
             sel_v, rw_v, xb, pos_v, pe_v, po_v, meta_v, cur_s, sx, sw, sm):
        wid = lax.axis_index("s") * info.num_cores + lax.axis_index("c")
        xcp = pltpu.async_copy(x3_hbm.at[pl.ds(wid * CT, CT)], xb, sx)
        pltpu.sync_copy(sel_hbm, sel_v)
        pltpu.sync_copy(rw_hbm.at[pl.ds(wid * CS, CS)], rw_v)

        zero = jnp.zeros((L,), jnp.int32)

        def scan8(b, accs):
            for q in range(CS // L):
                v = sel_v[pl.ds(b * CS + q * L, L)]
                accs = tuple(accs[e] + jnp.where(v == e, 1, 0)
                             for e in range(E))
            return accs

        pre = lax.fori_loop(0, wid, scan8, (zero,) * E)
        tot = lax.fori_loop(wid, NW, scan8, pre)

        pre_c = [jnp.sum(pre[e]) for e in range(E)]
        tot_c = [jnp.sum(tot[e]) for e in range(E)]
        run = 0
        for e in range(E):
            cur_s[e] = run + pre_c[e]
            run = run + tot_c[e]

        lane = lax.iota(jnp.int32, L)
        nvec = CS // L
        for p in range(nvec // 2):
            cs = [None] * (2 * L)
            for h in range(2):
                vi = 2 * p + h
                v = sel_v[pl.ds(wid * CS + vi * L, L)]
                for l in range(L):
                    e = v[l]
                    c = cur_s[e]
                    cur_s[e] = c + 1
                    cs[h * L + l] = c
            posv0 = jnp.zeros((L,), jnp.int32)
            posv1 = jnp.zeros((L,), jnp.int32)
            pev = jnp.zeros((L,), jnp.int32)
            pov = jnp.zeros((L,), jnp.int32)
            for l in range(L):
                posv0 = jnp.where(lane == l, cs[l], posv0)
                posv1 = jnp.where(lane == l, cs[L + l], posv1)
            for l in range(L):
                pev = jnp.where(lane == l, cs[2 * l], pev)
                pov = jnp.where(lane == l, cs[2 * l + 1], pov)
            pos_v[pl.ds((2 * p) * L, L)] = posv0
            pos_v[pl.ds((2 * p + 1) * L, L)] = posv1
            pe_v[pl.ds(p * L, L)] = pev
            po_v[pl.ds(p * L, L)] = pov

        xcp.wait()
        cp0 = pltpu.async_copy(xb, xs_hbm.at[pe_v], sx)
        cp1 = pltpu.async_copy(xb, xs_hbm.at[po_v], sw)
        cpw = pltpu.async_copy(rw_v, ws_hbm.at[pos_v], sm)
        pltpu.sync_copy(pe_v, pe_hbm.at[pl.ds(wid * CT, CT)])
        pltpu.sync_copy(po_v, po_hbm.at[pl.ds(wid * CT, CT)])

        @pl.when(wid == 0)
        def _():
            ftv = jnp.zeros((L,), jnp.int32)
            ntv = jnp.zeros((L,), jnp.int32)
            stv = jnp.zeros((L,), jnp.int32)
            env = jnp.zeros((L,), jnp.int32)
            run2 = 0
            for e in range(E):
                sz = tot_c[e]
                st = run2
                en = st + sz
                ft = jnp.where(sz > 0, st // TM_, 0)
                lt = jnp.where(sz > 0, (en - 1) // TM_, -1)
                nt = jnp.maximum(lt - ft + 1, 0)
                ftv = jnp.where(lane == e, ft, ftv)
                ntv = jnp.where(lane == e, nt, ntv)
                stv = jnp.where(lane == e, st, stv)
                env = jnp.where(lane == e, en, env)
                run2 = en
            meta_v[0, :] = ftv
            meta_v[1, :] = ntv
            meta_v[2, :] = stv
            meta_v[3, :] = env
            pltpu.sync_copy(meta_v, meta_hbm)

        cp0.wait()
        cp1.wait()
        cpw.wait()

    return prep


@functools.cache
def _make_combine(T, D):
    info = plsc.get_sparse_core_info()
    NW = info.num_cores * info.num_subcores
    CT = T // NW
    CH = 32
    NCH = CT // CH
    L = info.num_lanes
    mesh = plsc.VectorSubcoreMesh(core_axis_name="c", subcore_axis_name="s")

    @functools.partial(
        pl.kernel, mesh=mesh,
        out_type=jax.ShapeDtypeStruct((T, D), jnp.float32),
        scratch_types=[
            pltpu.VMEM((CH,), jnp.int32),
            pltpu.VMEM((CH,), jnp.int32),
            pltpu.VMEM((CH, D), jnp.float32),
            pltpu.VMEM((CH, D), jnp.float32),
            pltpu.VMEM((CH, D), jnp.float32),
            pltpu.SemaphoreType.DMA,
            pltpu.SemaphoreType.DMA,
        ])
    def combine(y_hbm, pe_hbm, po_hbm, out_hbm, i0, i1, b0, b1, ob, s0, s1):
        wid = lax.axis_index("s") * info.num_cores + lax.axis_index("c")
        base = wid * CT
        for c in range(NCH):
            tb = base + c * CH
            pltpu.sync_copy(pe_hbm.at[pl.ds(tb, CH)], i0)
            pltpu.sync_copy(po_hbm.at[pl.ds(tb, CH)], i1)
            cp0 = pltpu.async_copy(y_hbm.at[i0], b0, s0)
            cp1 = pltpu.async_copy(y_hbm.at[i1], b1, s1)
            cp0.wait()
            cp1.wait()

            def col(cc, carry):
                off = cc * L
                for j in range(CH):
                    ob[j, pl.ds(off, L)] = (b0[j, pl.ds(off, L)]
                                            + b1[j, pl.ds(off, L)])
                return carry

            lax.fori_loop(0, D // L, col, 0)
            pltpu.sync_copy(ob, out_hbm.at[pl.ds(tb, CH)])

    return combine


@functools.partial(jax.jit, static_argnums=())
def kernel(x_TD, router_weights_TX, selected_experts_TX,
           kernel_gating, kernel_up_proj, kernel_down_proj):
    T, D = x_TD.shape
    K = router_weights_TX.shape[1]
    E, _, F = kernel_gating.shape
    M = T * K
    m_tiles = M // TM
    NF = F // TF

    SL = 4
    flat = selected_experts_TX.reshape(-1).astype(jnp.int32)
    x_bf = x_TD.astype(jnp.bfloat16)
    x3 = jax.lax.bitcast_convert_type(
        x_bf.reshape(T, D // 2, 2), jnp.int32)
    xs3, w_flat, pe, po, meta = _make_prep(T, D, K, E, M, SL)(
        x3, flat, router_weights_TX.reshape(-1))
    x_sorted = jax.lax.bitcast_convert_type(
        xs3.reshape(M, D // 2)[..., None], jnp.bfloat16).reshape(M, D)
    w_sorted = w_flat[:, None]
    first_tile = meta[0, :E]
    ntiles = meta[1, :E]
    starts = meta[2, :E]
    ends = meta[3, :E]

    grid_spec = pltpu.PrefetchScalarGridSpec(
        num_scalar_prefetch=4,
        grid=(E, NF),
        in_specs=[
            pl.BlockSpec((M, D), lambda e, f, ft, nt, st, en: (0, 0)),
            pl.BlockSpec((M, 1), lambda e, f, ft, nt, st, en: (0, 0)),
            pl.BlockSpec((1, D, TF), lambda e, f, ft, nt, st, en: (e, 0, f)),
            pl.BlockSpec((1, D, TF), lambda e, f, ft, nt, st, en: (e, 0, f)),
            pl.BlockSpec((1, TF, D), lambda e, f, ft, nt, st, en: (e, f, 0)),
        ],
        out_specs=pl.BlockSpec((M, D), lambda e, f, ft, nt, st, en: (0, 0)),
        scratch_shapes=[
            pltpu.VMEM((D, TF), jnp.bfloat16),
            pltpu.VMEM((D, TF), jnp.bfloat16),
            pltpu.VMEM((TF, D), jnp.bfloat16),
        ],
    )
    y_sorted = pl.pallas_call(
        _fused_moe_body,
        grid_spec=grid_spec,
        out_shape=jax.ShapeDtypeStruct((M, D), jnp.float32),
        compiler_params=pltpu.CompilerParams(vmem_limit_bytes=62 * 1024 * 1024),
    )(first_tile, ntiles, starts, ends,
      x_sorted, w_sorted, kernel_gating, kernel_up_proj, kernel_down_proj)

    out_TD = _make_combine(T, D)(y_sorted, pe, po)
    return out_TD.astype(jnp.float32)


# speedup vs baseline: 1.4014x; 1.4014x over previous
import functools

import jax
import jax.numpy as jnp
from jax import lax
from jax.experimental import pallas as pl
from jax.experimental.pallas import tpu as pltpu
from jax.experimental.pallas import tpu_sc as plsc


TM = 128
TF = 1024


def _fused_moe_body(ft_ref, nt_ref, st_ref, en_ref,
                    x_ref, w_ref, wg_ref, wu_ref, wd_ref, out_ref,
                    wg_bf, wu_bf, wd_bf):
    e = pl.program_id(0)
    f = pl.program_id(1)

    @pl.when((e == 0) & (f == 0))
    def _():
        out_ref[...] = jnp.zeros_like(out_ref)

    wg_bf[...] = wg_ref[0].astype(jnp.bfloat16)
    wu_bf[...] = wu_ref[0].astype(jnp.bfloat16)
    wd_bf[...] = wd_ref[0].astype(jnp.bfloat16)

    start = st_ref[e]
    end = en_ref[e]
    first = ft_ref[e]

    def chunk(c, carry):
        base = (first + c) * TM
        x = x_ref[pl.ds(base, TM), :]
        gate = jnp.dot(x, wg_bf[...], preferred_element_type=jnp.float32)
        up = jnp.dot(x, wu_bf[...], preferred_element_type=jnp.float32)
        fused = gate * jax.lax.logistic(gate) * up
        fused = fused * w_ref[pl.ds(base, TM), :]
        part = jnp.dot(fused.astype(jnp.bfloat16), wd_bf[...],
                       preferred_element_type=jnp.float32)
        row = base + jax.lax.broadcasted_iota(jnp.int32, (TM, 1), 0)
        mask = (row >= start) & (row < end)
        out_ref[pl.ds(base, TM), :] += jnp.where(mask, part, 0.0)
        return carry

    jax.lax.fori_loop(0, nt_ref[e], chunk, 0)


@functools.cache
def _make_combine(T, D):
    info = plsc.get_sparse_core_info()
    NW = info.num_cores * info.num_subcores
    CT = T // NW
    CH = 32
    NCH = CT // CH
    L = info.num_lanes
    mesh = plsc.VectorSubcoreMesh(core_axis_name="c", subcore_axis_name="s")

    @functools.partial(
        pl.kernel, mesh=mesh,
        out_type=jax.ShapeDtypeStruct((T, D), jnp.float32),
        scratch_types=[
            pltpu.VMEM((CH,), jnp.int32),
            pltpu.VMEM((CH,), jnp.int32),
            pltpu.VMEM((CH, D), jnp.float32),
            pltpu.VMEM((CH, D), jnp.float32),
            pltpu.VMEM((CH, D), jnp.float32),
            pltpu.SemaphoreType.DMA,
            pltpu.SemaphoreType.DMA,
        ])
    def combine(y_hbm, pe_hbm, po_hbm, out_hbm, i0, i1, b0, b1, ob, s0, s1):
        wid = lax.axis_index("s") * info.num_cores + lax.axis_index("c")
        base = wid * CT
        for c in range(NCH):
            tb = base + c * CH
            pltpu.sync_copy(pe_hbm.at[pl.ds(tb, CH)], i0)
            pltpu.sync_copy(po_hbm.at[pl.ds(tb, CH)], i1)
            cp0 = pltpu.async_copy(y_hbm.at[i0], b0, s0)
            cp1 = pltpu.async_copy(y_hbm.at[i1], b1, s1)
            cp0.wait()
            cp1.wait()

            def col(cc, carry):
                off = cc * L
                for j in range(CH):
                    ob[j, pl.ds(off, L)] = (b0[j, pl.ds(off, L)]
                                            + b1[j, pl.ds(off, L)])
                return carry

            lax.fori_loop(0, D // L, col, 0)
            pltpu.sync_copy(ob, out_hbm.at[pl.ds(tb, CH)])

    return combine


@functools.partial(jax.jit, static_argnums=())
def kernel(x_TD, router_weights_TX, selected_experts_TX,
           kernel_gating, kernel_up_proj, kernel_down_proj):
    T, D = x_TD.shape
    K = router_weights_TX.shape[1]
    E, _, F = kernel_gating.shape
    M = T * K
    m_tiles = M // TM
    NF = F // TF

    flat = selected_experts_TX.reshape(-1)
    oh = (flat[:, None] == jnp.arange(E)[None, :]).astype(jnp.int32)
    csum = jnp.cumsum(oh, axis=0)
    sizes = csum[-1]
    ends = jnp.cumsum(sizes)
    starts = ends - sizes
    rank = jnp.sum(oh * csum, axis=1) - 1
    pos = jnp.sum(oh * starts[None, :], axis=1) + rank

    slot_iota = jnp.arange(M, dtype=jnp.int32)
    tok_sorted = jnp.zeros((M,), jnp.int32).at[pos].set(slot_iota // K)
    x_sorted = jnp.take(x_TD.astype(jnp.bfloat16), tok_sorted, axis=0)
    w_sorted = jnp.zeros((M,), jnp.float32).at[pos].set(
        router_weights_TX.reshape(-1))[:, None]

    nonempty = sizes > 0
    first_tile = jnp.where(nonempty, starts // TM, 0).astype(jnp.int32)
    last_tile = jnp.where(nonempty, (ends - 1) // TM, -1)
    ntiles = jnp.maximum(last_tile - first_tile + 1, 0).astype(jnp.int32)

    grid_spec = pltpu.PrefetchScalarGridSpec(
        num_scalar_prefetch=4,
        grid=(E, NF),
        in_specs=[
            pl.BlockSpec((M, D), lambda e, f, ft, nt, st, en: (0, 0)),
            pl.BlockSpec((M, 1), lambda e, f, ft, nt, st, en: (0, 0)),
            pl.BlockSpec((1, D, TF), lambda e, f, ft, nt, st, en: (e, 0, f)),
            pl.BlockSpec((1, D, TF), lambda e, f, ft, nt, st, en: (e, 0, f)),
            pl.BlockSpec((1, TF, D), lambda e, f, ft, nt, st, en: (e, f, 0)),
        ],
        out_specs=pl.BlockSpec((M, D), lambda e, f, ft, nt, st, en: (0, 0)),
        scratch_shapes=[
            pltpu.VMEM((D, TF), jnp.bfloat16),
            pltpu.VMEM((D, TF), jnp.bfloat16),
            pltpu.VMEM((TF, D), jnp.bfloat16),
        ],
    )
    y_sorted = pl.pallas_call(
        _fused_moe_body,
        grid_spec=grid_spec,
        out_shape=jax.ShapeDtypeStruct((M, D), jnp.float32),
        compiler_params=pltpu.CompilerParams(vmem_limit_bytes=62 * 1024 * 1024),
    )(first_tile, ntiles, starts.astype(jnp.int32), ends.astype(jnp.int32),
      x_sorted, w_sorted, kernel_gating, kernel_up_proj, kernel_down_proj)

    pos_TK = pos.reshape(T, K).astype(jnp.int32)
    out_TD = _make_combine(T, D)(y_sorted, pos_TK[:, 0], pos_TK[:, 1])
    return out_TD.astype(jnp.float32)


# speedup vs baseline: 1.5035x; 1.0729x over previous
import functools

import jax
import jax.numpy as jnp
from jax import lax
from jax.experimental import pallas as pl
from jax.experimental.pallas import tpu as pltpu
from jax.experimental.pallas import tpu_sc as plsc


TM = 128
TF = 1024


def _fused_moe_body(ft_ref, nt_ref, st_ref, en_ref,
                    x_ref, wg_ref, wu_ref, wd_ref, out_ref,
                    wg_bf, wu_bf, wd_bf):
    e = pl.program_id(0)
    f = pl.program_id(1)

    @pl.when((e == 0) & (f == 0))
    def _():
        out_ref[...] = jnp.zeros_like(out_ref)

    wg_bf[...] = wg_ref[0].astype(jnp.bfloat16)
    wu_bf[...] = wu_ref[0].astype(jnp.bfloat16)
    wd_bf[...] = wd_ref[0].astype(jnp.bfloat16)

    start = st_ref[e]
    end = en_ref[e]
    first = ft_ref[e]

    def chunk(c, carry):
        base = (first + c) * TM
        x = x_ref[pl.ds(base, TM), :]
        gate = jnp.dot(x, wg_bf[...], preferred_element_type=jnp.float32)
        up = jnp.dot(x, wu_bf[...], preferred_element_type=jnp.float32)
        fused = gate * jax.lax.logistic(gate) * up
        part = jnp.dot(fused.astype(jnp.bfloat16), wd_bf[...],
                       preferred_element_type=jnp.float32)
        row = base + jax.lax.broadcasted_iota(jnp.int32, (TM, 1), 0)
        mask = (row >= start) & (row < end)
        out_ref[pl.ds(base, TM), :] += jnp.where(mask, part, 0.0)
        return carry

    jax.lax.fori_loop(0, nt_ref[e], chunk, 0)


@functools.cache
def _make_combine(T, D):
    info = plsc.get_sparse_core_info()
    NW = info.num_cores * info.num_subcores
    CT = T // NW
    CH = 32
    NCH = CT // CH
    L = info.num_lanes
    mesh = plsc.VectorSubcoreMesh(core_axis_name="c", subcore_axis_name="s")

    @functools.partial(
        pl.kernel, mesh=mesh,
        out_type=jax.ShapeDtypeStruct((T, D), jnp.float32),
        scratch_types=[
            pltpu.VMEM((CH,), jnp.int32),
            pltpu.VMEM((CH,), jnp.int32),
            pltpu.VMEM((2 * CH,), jnp.float32),
            pltpu.VMEM((CH, D), jnp.float32),
            pltpu.VMEM((CH, D), jnp.float32),
            pltpu.VMEM((CH, D), jnp.float32),
            pltpu.SemaphoreType.DMA,
            pltpu.SemaphoreType.DMA,
        ])
    def combine(y_hbm, pe_hbm, po_hbm, rw_hbm, out_hbm,
                i0, i1, wb, b0, b1, ob, s0, s1):
        wid = lax.axis_index("s") * info.num_cores + lax.axis_index("c")
        base = wid * CT
        for c in range(NCH):
            tb = base + c * CH
            pltpu.sync_copy(pe_hbm.at[pl.ds(tb, CH)], i0)
            pltpu.sync_copy(po_hbm.at[pl.ds(tb, CH)], i1)
            pltpu.sync_copy(rw_hbm.at[pl.ds(2 * tb, 2 * CH)], wb)
            cp0 = pltpu.async_copy(y_hbm.at[i0], b0, s0)
            cp1 = pltpu.async_copy(y_hbm.at[i1], b1, s1)
            cp0.wait()
            cp1.wait()

            ws = []
            for g in range(2 * CH // L):
                wv = wb[pl.ds(g * L, L)]
                for l in range(L):
                    ws.append(wv[l])

            def col(cc, carry):
                off = cc * L
                for j in range(CH):
                    ob[j, pl.ds(off, L)] = (
                        b0[j, pl.ds(off, L)] * ws[2 * j]
                        + b1[j, pl.ds(off, L)] * ws[2 * j + 1])
                return carry

            lax.fori_loop(0, D // L, col, 0)
            pltpu.sync_copy(ob, out_hbm.at[pl.ds(tb, CH)])

    return combine


@functools.partial(jax.jit, static_argnums=())
def kernel(x_TD, router_weights_TX, selected_experts_TX,
           kernel_gating, kernel_up_proj, kernel_down_proj):
    T, D = x_TD.shape
    K = router_weights_TX.shape[1]
    E, _, F = kernel_gating.shape
    M = T * K
    m_tiles = M // TM
    NF = F // TF

    flat = selected_experts_TX.reshape(-1)
    oh = (flat[:, None] == jnp.arange(E)[None, :]).astype(jnp.int32)
    csum = jnp.cumsum(oh, axis=0)
    sizes = csum[-1]
    ends = jnp.cumsum(sizes)
    starts = ends - sizes
    rank = jnp.sum(oh * csum, axis=1) - 1
    pos = jnp.sum(oh * starts[None, :], axis=1) + rank

    slot_iota = jnp.arange(M, dtype=jnp.int32)
    tok_sorted = jnp.zeros((M,), jnp.int32).at[pos].set(slot_iota // K)
    x_sorted = jnp.take(x_TD.astype(jnp.bfloat16), tok_sorted, axis=0)

    nonempty = sizes > 0
    first_tile = jnp.where(nonempty, starts // TM, 0).astype(jnp.int32)
    last_tile = jnp.where(nonempty, (ends - 1) // TM, -1)
    ntiles = jnp.maximum(last_tile - first_tile + 1, 0).astype(jnp.int32)

    grid_spec = pltpu.PrefetchScalarGridSpec(
        num_scalar_prefetch=4,
        grid=(E, NF),
        in_specs=[
            pl.BlockSpec((M, D), lambda e, f, ft, nt, st, en: (0, 0)),
            pl.BlockSpec((1, D, TF), lambda e, f, ft, nt, st, en: (e, 0, f)),
            pl.BlockSpec((1, D, TF), lambda e, f, ft, nt, st, en: (e, 0, f)),
            pl.BlockSpec((1, TF, D), lambda e, f, ft, nt, st, en: (e, f, 0)),
        ],
        out_specs=pl.BlockSpec((M, D), lambda e, f, ft, nt, st, en: (0, 0)),
        scratch_shapes=[
            pltpu.VMEM((D, TF), jnp.bfloat16),
            pltpu.VMEM((D, TF), jnp.bfloat16),
            pltpu.VMEM((TF, D), jnp.bfloat16),
        ],
    )
    y_sorted = pl.pallas_call(
        _fused_moe_body,
        grid_spec=grid_spec,
        out_shape=jax.ShapeDtypeStruct((M, D), jnp.float32),
        compiler_params=pltpu.CompilerParams(vmem_limit_bytes=62 * 1024 * 1024),
    )(first_tile, ntiles, starts.astype(jnp.int32), ends.astype(jnp.int32),
      x_sorted, kernel_gating, kernel_up_proj, kernel_down_proj)

    pos_TK = pos.reshape(T, K).astype(jnp.int32)
    out_TD = _make_combine(T, D)(y_sorted, pos_TK[:, 0], pos_TK[:, 1],
                                 router_weights_TX.reshape(-1))
    return out_TD.astype(jnp.float32)


# speedup vs baseline: 1.5037x; 1.0001x over previous
import functools

import jax
import jax.numpy as jnp
from jax import lax
from jax.experimental import pallas as pl
from jax.experimental.pallas import tpu as pltpu
from jax.experimental.pallas import tpu_sc as plsc


TM = 128
TF = 1024


def _fused_moe_body(ft_ref, nt_ref, st_ref, en_ref,
                    x_ref, wg_ref, wu_ref, wd_ref, out_ref,
                    wg_bf, wu_bf, wd_bf):
    e = pl.program_id(0)
    f = pl.program_id(1)

    @pl.when((e == 0) & (f == 0))
    def _():
        out_ref[...] = jnp.zeros_like(out_ref)

    wg_bf[...] = wg_ref[0].astype(jnp.bfloat16)
    wu_bf[...] = wu_ref[0].astype(jnp.bfloat16)
    wd_bf[...] = wd_ref[0].astype(jnp.bfloat16)

    start = st_ref[e]
    end = en_ref[e]
    first = ft_ref[e]

    def chunk(c, carry):
        base = (first + c) * TM
        x = x_ref[pl.ds(base, TM), :]
        gate = jnp.dot(x, wg_bf[...], preferred_element_type=jnp.float32)
        up = jnp.dot(x, wu_bf[...], preferred_element_type=jnp.float32)
        fused = gate * jax.lax.logistic(gate) * up
        part = jnp.dot(fused.astype(jnp.bfloat16), wd_bf[...],
                       preferred_element_type=jnp.float32)
        row = base + jax.lax.broadcasted_iota(jnp.int32, (TM, 1), 0)
        mask = (row >= start) & (row < end)
        out_ref[pl.ds(base, TM), :] += jnp.where(mask, part, 0.0)
        return carry

    jax.lax.fori_loop(0, nt_ref[e], chunk, 0)


@functools.cache
def _make_combine(T, D):
    info = plsc.get_sparse_core_info()
    NW = info.num_cores * info.num_subcores
    CT = T // NW
    CH = 32
    NCH = CT // CH
    L = info.num_lanes
    mesh = plsc.VectorSubcoreMesh(core_axis_name="c", subcore_axis_name="s")

    @functools.partial(
        pl.kernel, mesh=mesh,
        out_type=jax.ShapeDtypeStruct((T, D), jnp.float32),
        scratch_types=[
            pltpu.VMEM((CH,), jnp.int32),
            pltpu.VMEM((CH,), jnp.int32),
            pltpu.VMEM((2 * CH,), jnp.float32),
            pltpu.VMEM((CH, D), jnp.float32),
            pltpu.VMEM((CH, D), jnp.float32),
            pltpu.VMEM((CH, D), jnp.float32),
            pltpu.SemaphoreType.DMA,
            pltpu.SemaphoreType.DMA,
        ])
    def combine(y_hbm, pe_hbm, po_hbm, rw_hbm, out_hbm,
                i0, i1, wb, b0, b1, ob, s0, s1):
        wid = lax.axis_index("s") * info.num_cores + lax.axis_index("c")
        base = wid * CT
        for c in range(NCH):
            tb = base + c * CH
            pltpu.sync_copy(pe_hbm.at[pl.ds(tb, CH)], i0)
            pltpu.sync_copy(po_hbm.at[pl.ds(tb, CH)], i1)
            pltpu.sync_copy(rw_hbm.at[pl.ds(2 * tb, 2 * CH)], wb)
            cp0 = pltpu.async_copy(y_hbm.at[i0], b0, s0)
            cp1 = pltpu.async_copy(y_hbm.at[i1], b1, s1)
            cp0.wait()
            cp1.wait()

            ws = []
            for g in range(2 * CH // L):
                wv = wb[pl.ds(g * L, L)]
                for l in range(L):
                    ws.append(wv[l])

            def col(cc, carry):
                off = cc * L
                for j in range(CH):
                    ob[j, pl.ds(off, L)] = (
                        b0[j, pl.ds(off, L)] * ws[2 * j]
                        + b1[j, pl.ds(off, L)] * ws[2 * j + 1])
                return carry

            lax.fori_loop(0, D // L, col, 0)
            pltpu.sync_copy(ob, out_hbm.at[pl.ds(tb, CH)])

    return combine


@functools.partial(jax.jit, static_argnums=())
def kernel(x_TD, router_weights_TX, selected_experts_TX,
           kernel_gating, kernel_up_proj, kernel_down_proj):
    T, D = x_TD.shape
    K = router_weights_TX.shape[1]
    E, _, F = kernel_gating.shape
    M = T * K
    m_tiles = M // TM
    NF = F // TF

    flat = selected_experts_TX.reshape(-1)
    oh = (jnp.arange(E)[:, None] == flat[None, :]).astype(jnp.int32)
    csum = jnp.cumsum(oh, axis=1)
    sizes = csum[:, -1]
    ends = jnp.cumsum(sizes)
    starts = ends - sizes
    rank = jnp.sum(oh * csum, axis=0) - 1
    pos = jnp.sum(oh * starts[:, None], axis=0) + rank

    slot_iota = jnp.arange(M, dtype=jnp.int32)
    tok_sorted = jnp.zeros((M,), jnp.int32).at[pos].set(slot_iota // K)
    x_sorted = jnp.take(x_TD.astype(jnp.bfloat16), tok_sorted, axis=0)

    nonempty = sizes > 0
    first_tile = jnp.where(nonempty, starts // TM, 0).astype(jnp.int32)
    last_tile = jnp.where(nonempty, (ends - 1) // TM, -1)
    ntiles = jnp.maximum(last_tile - first_tile + 1, 0).astype(jnp.int32)

    grid_spec = pltpu.PrefetchScalarGridSpec(
        num_scalar_prefetch=4,
        grid=(E, NF),
        in_specs=[
            pl.BlockSpec((M, D), lambda e, f, ft, nt, st, en: (0, 0)),
            pl.BlockSpec((1, D, TF), lambda e, f, ft, nt, st, en: (e, 0, f)),
            pl.BlockSpec((1, D, TF), lambda e, f, ft, nt, st, en: (e, 0, f)),
            pl.BlockSpec((1, TF, D), lambda e, f, ft, nt, st, en: (e, f, 0)),
        ],
        out_specs=pl.BlockSpec((M, D), lambda e, f, ft, nt, st, en: (0, 0)),
        scratch_shapes=[
            pltpu.VMEM((D, TF), jnp.bfloat16),
            pltpu.VMEM((D, TF), jnp.bfloat16),
            pltpu.VMEM((TF, D), jnp.bfloat16),
        ],
    )
    y_sorted = pl.pallas_call(
        _fused_moe_body,
        grid_spec=grid_spec,
        out_shape=jax.ShapeDtypeStruct((M, D), jnp.float32),
        compiler_params=pltpu.CompilerParams(vmem_limit_bytes=62 * 1024 * 1024),
    )(first_tile, ntiles, starts.astype(jnp.int32), ends.astype(jnp.int32),
      x_sorted, kernel_gating, kernel_up_proj, kernel_down_proj)

    pos_TK = pos.reshape(T, K).astype(jnp.int32)
    out_TD = _make_combine(T, D)(y_sorted, pos_TK[:, 0], pos_TK[:, 1],
                                 router_weights_TX.reshape(-1))
    return out_TD.astype(jnp.float32)
